# Initial kernel scaffold; baseline (speedup 1.0000x reference)
import jax
import jax.numpy as jnp
from jax.experimental import pallas as pl


def kernel(global_img_feat, global_text_feat, local_img_feats, local_text_feats, centers, text_centers, memory_feats, position, memory_vid):
    raise NotImplementedError("write your pallas kernel here")


# speedup vs baseline: 3.2716x; 3.2716x over previous
import jax
import jax.numpy as jnp
from jax import lax
from jax.experimental import pallas as pl
from jax.experimental.pallas import tpu as pltpu

_SCALE = 10.0
_KTOP = 10
_TEMP = 0.5
_NEG_INF = -1e30


def _l2n(x):
    n = jnp.sqrt(jnp.sum(x * x, axis=-1, keepdims=True))
    return x / jnp.maximum(n, 1e-12)


def _k1_body(gif_ref, gtf_ref, ltf_ref, mem_ref, sim_ref, align_ref):
    B = gif_ref.shape[0]
    img = _l2n(gif_ref[...])
    txt = _l2n(gtf_ref[...])
    diag = (lax.broadcasted_iota(jnp.int32, (B, B), 0)
            == lax.broadcasted_iota(jnp.int32, (B, B), 1))

    def sim_logits(x):
        xb = x.astype(jnp.bfloat16)
        s = lax.dot_general(xb, xb, (((1,), (1,)), ((), ())),
                            preferred_element_type=jnp.float32) / _TEMP
        return jnp.where(diag, _NEG_INF, s)

    img_sim = sim_logits(img)
    txt_sim = sim_logits(txt)

    def log_softmax(s):
        z = s - jnp.max(s, axis=1, keepdims=True)
        return z - jnp.log(jnp.sum(jnp.exp(z), axis=1, keepdims=True))

    img_logp = log_softmax(img_sim)
    txt_logp = log_softmax(txt_sim)
    img_p = jnp.exp(img_logp)
    txt_p = jnp.exp(txt_logp)

    def kl(logp, p):
        return jnp.sum(p * (jnp.log(jnp.maximum(p, 1e-12)) - logp)) / B

    align_ref[...] = jnp.reshape(
        0.5 * (kl(img_logp, txt_p) + kl(txt_logp, img_p)), (1, 1))

    tmean = _l2n(jnp.mean(ltf_ref[...], axis=0)).astype(jnp.bfloat16)
    memn = _l2n(mem_ref[...]).astype(jnp.bfloat16)
    sim_ref[...] = lax.dot_general(tmean, memn, (((1,), (1,)), ((), ())),
                                   preferred_element_type=jnp.float32)


def _k2_body(sim_ref, vid_ref, pos_ref, idx_ref, pv_ref, mask_ref):
    B, M = sim_ref.shape
    miota = lax.broadcasted_iota(jnp.int32, (B, M), 1)
    cur = sim_ref[...]
    vidf = vid_ref[...]
    h_sum = jnp.zeros((B, M), jnp.float32)
    idx_cols = []
    pv_cols = []
    for _ in range(_KTOP):
        v = jnp.max(cur, axis=1, keepdims=True)
        picked = cur == v
        idxk = jnp.min(jnp.where(picked, miota, M), axis=1, keepdims=True)
        hit = miota == idxk
        cur = jnp.where(hit, _NEG_INF, cur)
        hitf = hit.astype(jnp.float32)
        h_sum = h_sum + hitf
        idx_cols.append(idxk)
        pv_cols.append(jnp.sum(hitf * vidf, axis=1, keepdims=True))
    pad_i = jnp.zeros((B, 16 - _KTOP), jnp.int32)
    idx_ref[...] = jnp.concatenate(idx_cols + [pad_i], axis=1)
    pv_ref[...] = jnp.concatenate(
        [c.astype(jnp.int32) for c in pv_cols] + [pad_i], axis=1)
    any_pick = jnp.sum(h_sum, axis=0, keepdims=True)
    pos_b = pos_ref[...]
    pos_hit = jnp.sum((miota == pos_b).astype(jnp.float32), axis=0,
                      keepdims=True)
    mask_ref[...] = jnp.where((any_pick + pos_hit) > 0.0, 0.0, 1.0)


def _k3_body(lif_ref, cen_ref, mask_ref, idx_ref, align_ref,
             total_ref, local_ref, spos_ref, sneg_ref, acc_ref):
    p = pl.program_id(0)
    mb = pl.program_id(1)
    num_p = pl.num_programs(0)
    num_mb = pl.num_programs(1)

    @pl.when(jnp.logical_and(p == 0, mb == 0))
    def _():
        acc_ref[...] = jnp.zeros_like(acc_ref)

    @pl.when(mb == 0)
    def _():
        spos_ref[...] = jnp.zeros_like(spos_ref)
        sneg_ref[...] = jnp.zeros_like(sneg_ref)

    pf = lif_ref[0]
    cb = cen_ref[0]
    B = pf.shape[0]
    blk = cb.shape[0]
    a2 = jnp.sum(pf * pf, axis=1, keepdims=True)
    c2 = jnp.sum(cb * cb, axis=1, keepdims=True)
    dot = lax.dot_general(pf.astype(jnp.bfloat16), cb.astype(jnp.bfloat16),
                          (((1,), (1,)), ((), ())),
                          preferred_element_type=jnp.float32)
    d2 = a2 + c2.T - 2.0 * dot
    dist = jnp.sqrt(jnp.maximum(d2, 1e-12))
    e = jnp.exp(-_SCALE * dist)

    base = mb * blk
    col = base + lax.broadcasted_iota(jnp.int32, (B, blk), 1)
    h = jnp.zeros((B, blk), jnp.float32)
    for k in range(_KTOP):
        h = h + (col == idx_ref[:, k:k + 1]).astype(jnp.float32)

    spos_ref[...] += jnp.sum(e * h, axis=1, keepdims=True).T
    sneg_ref[...] += jnp.sum(e * mask_ref[...], axis=1, keepdims=True).T

    @pl.when(mb == num_mb - 1)
    def _():
        x = jnp.log(spos_ref[...])
        y = jnp.log(sneg_ref[...])
        l = jnp.sum(y - x) / B
        l = jnp.where(jnp.isnan(l), 0.0, l)
        acc_ref[...] += jnp.reshape(l, (1, 1))

        @pl.when(p == num_p - 1)
        def _():
            loc = acc_ref[...] / num_p
            local_ref[...] = loc
            total_ref[...] = loc + 0.5 * align_ref[...]


def _impl(global_img_feat, global_text_feat, local_img_feats,
          local_text_feats, centers, text_centers, memory_feats,
          position, memory_vid, interpret=False):
    B, d = global_img_feat.shape
    P, M, _ = centers.shape
    vidf = memory_vid.astype(jnp.float32).reshape(1, M)
    pos2d = position.reshape(B, 1)

    sim, align = pl.pallas_call(
        _k1_body,
        out_shape=[jax.ShapeDtypeStruct((B, M), jnp.float32),
                   jax.ShapeDtypeStruct((1, 1), jnp.float32)],
        interpret=interpret,
    )(global_img_feat, global_text_feat, local_text_feats, memory_feats)

    idx, pv, mask = pl.pallas_call(
        _k2_body,
        out_shape=[jax.ShapeDtypeStruct((B, 16), jnp.int32),
                   jax.ShapeDtypeStruct((B, 16), jnp.int32),
                   jax.ShapeDtypeStruct((1, M), jnp.float32)],
        interpret=interpret,
    )(sim, vidf, pos2d)

    nblk = 8
    blk = M // nblk
    total, local = pl.pallas_call(
        _k3_body,
        grid=(P, nblk),
        in_specs=[
            pl.BlockSpec((1, B, d), lambda p, m: (p, 0, 0)),
            pl.BlockSpec((1, blk, d), lambda p, m: (p, m, 0)),
            pl.BlockSpec((1, blk), lambda p, m: (0, m)),
            pl.BlockSpec((B, 16), lambda p, m: (0, 0)),
            pl.BlockSpec((1, 1), lambda p, m: (0, 0)),
        ],
        out_specs=[
            pl.BlockSpec((1, 1), lambda p, m: (0, 0)),
            pl.BlockSpec((1, 1), lambda p, m: (0, 0)),
        ],
        out_shape=[jax.ShapeDtypeStruct((1, 1), jnp.float32),
                   jax.ShapeDtypeStruct((1, 1), jnp.float32)],
        scratch_shapes=[pltpu.VMEM((1, B), jnp.float32),
                        pltpu.VMEM((1, B), jnp.float32),
                        pltpu.VMEM((1, 1), jnp.float32)],
        interpret=interpret,
    )(local_img_feats, centers, mask, idx, align)

    return (total[0, 0], local[0, 0], align[0, 0], pv[:, :_KTOP])


def kernel(global_img_feat, global_text_feat, local_img_feats,
           local_text_feats, centers, text_centers, memory_feats,
           position, memory_vid):
    return _impl(global_img_feat, global_text_feat, local_img_feats,
                 local_text_feats, centers, text_centers, memory_feats,
                 position, memory_vid)


# speedup vs baseline: 3.4188x; 1.0450x over previous
import jax
import jax.numpy as jnp
from jax import lax
from jax.experimental import pallas as pl
from jax.experimental.pallas import tpu as pltpu

_SCALE = 10.0
_KTOP = 10
_TEMP = 0.5
_NEG_INF = -1e30


def _l2n(x):
    n = jnp.sqrt(jnp.sum(x * x, axis=-1, keepdims=True))
    return x / jnp.maximum(n, 1e-12)


def _bdot(a, b):
    return lax.dot_general(a.astype(jnp.bfloat16), b.astype(jnp.bfloat16),
                           (((1,), (1,)), ((), ())),
                           preferred_element_type=jnp.float32)


def _k1_body(gif_ref, gtf_ref, ltf_ref, mem_ref, sim_ref, align_ref, tm_ref):
    m = pl.program_id(0)

    @pl.when(m == 0)
    def _():
        B = gif_ref.shape[0]
        img = _l2n(gif_ref[...])
        txt = _l2n(gtf_ref[...])
        diag = (lax.broadcasted_iota(jnp.int32, (B, B), 0)
                == lax.broadcasted_iota(jnp.int32, (B, B), 1))

        def sim_logits(x):
            return jnp.where(diag, _NEG_INF, _bdot(x, x) / _TEMP)

        img_sim = sim_logits(img)
        txt_sim = sim_logits(txt)

        def log_softmax(s):
            z = s - jnp.max(s, axis=1, keepdims=True)
            return z - jnp.log(jnp.sum(jnp.exp(z), axis=1, keepdims=True))

        img_logp = log_softmax(img_sim)
        txt_logp = log_softmax(txt_sim)
        img_p = jnp.exp(img_logp)
        txt_p = jnp.exp(txt_logp)

        def kl(logp, p):
            return jnp.sum(p * (jnp.log(jnp.maximum(p, 1e-12)) - logp)) / B

        align_ref[...] = jnp.reshape(
            0.5 * (kl(img_logp, txt_p) + kl(txt_logp, img_p)), (1, 1))
        tm_ref[...] = _l2n(jnp.mean(ltf_ref[...], axis=0))

    sim_ref[...] = _bdot(tm_ref[...], _l2n(mem_ref[...]))


def _k2_body(sim_ref, vid_ref, pos_ref, idx_ref, pv_ref, mask_ref):
    B, M = sim_ref.shape
    miota = lax.broadcasted_iota(jnp.int32, (B, M), 1)
    cur = sim_ref[...]
    idx_cols = []
    for _ in range(_KTOP):
        v = jnp.max(cur, axis=1, keepdims=True)
        idxk = jnp.min(jnp.where(cur == v, miota, M), axis=1, keepdims=True)
        idx_cols.append(idxk)
        cur = jnp.where(miota == idxk, _NEG_INF, cur)
    vidf = vid_ref[...]
    any_pick = jnp.zeros((1, M), jnp.float32)
    pv_cols = []
    for k in range(_KTOP):
        hk = (miota == idx_cols[k]).astype(jnp.float32)
        any_pick = any_pick + jnp.sum(hk, axis=0, keepdims=True)
        pv_cols.append(jnp.sum(hk * vidf, axis=1, keepdims=True))
    pad_i = jnp.zeros((B, 16 - _KTOP), jnp.int32)
    idx_ref[...] = jnp.concatenate(idx_cols + [pad_i], axis=1)
    pv_ref[...] = jnp.concatenate(
        [c.astype(jnp.int32) for c in pv_cols] + [pad_i], axis=1)
    pos_hit = jnp.sum((miota == pos_ref[...]).astype(jnp.float32), axis=0,
                      keepdims=True)
    mask_ref[...] = jnp.where((any_pick + pos_hit) > 0.0, 0.0, 1.0)


def _k3_body(lif_ref, cen_ref, mask_ref, idx_ref, align_ref,
             total_ref, local_ref, spos_ref, sneg_ref, h_ref):
    mb = pl.program_id(0)
    p = pl.program_id(1)
    num_mb = pl.num_programs(0)
    num_p = pl.num_programs(1)

    pf = lif_ref[0]
    cb = cen_ref[0]
    B = pf.shape[0]
    blk = cb.shape[0]

    @pl.when(jnp.logical_and(mb == 0, p == 0))
    def _():
        spos_ref[...] = jnp.zeros_like(spos_ref)
        sneg_ref[...] = jnp.zeros_like(sneg_ref)

    @pl.when(p == 0)
    def _():
        base = mb * blk
        col = base + lax.broadcasted_iota(jnp.int32, (B, blk), 1)
        h = jnp.zeros((B, blk), jnp.float32)
        for k in range(_KTOP):
            h = h + (col == idx_ref[:, k:k + 1]).astype(jnp.float32)
        h_ref[...] = h

    a2 = jnp.sum(pf * pf, axis=1, keepdims=True)
    c2 = jnp.sum(cb * cb, axis=1, keepdims=True)
    d2 = a2 + c2.T - 2.0 * _bdot(pf, cb)
    dist = jnp.sqrt(jnp.maximum(d2, 1e-12))
    e = jnp.exp(-_SCALE * dist)

    lanes = pl.ds(p * B, B)
    spos_ref[:, lanes] += jnp.sum(e * h_ref[...], axis=1, keepdims=True).T
    sneg_ref[:, lanes] += jnp.sum(e * mask_ref[...], axis=1, keepdims=True).T

    @pl.when(jnp.logical_and(mb == num_mb - 1, p == num_p - 1))
    def _():
        acc = jnp.zeros((1, 1), jnp.float32)
        for q in range(4):
            x = jnp.log(spos_ref[:, q * B:(q + 1) * B])
            y = jnp.log(sneg_ref[:, q * B:(q + 1) * B])
            l = jnp.sum(y - x) / B
            l = jnp.where(jnp.isnan(l), 0.0, l)
            acc = acc + jnp.reshape(l, (1, 1))
        loc = acc / num_p
        local_ref[...] = loc
        total_ref[...] = loc + 0.5 * align_ref[...]


def _impl(global_img_feat, global_text_feat, local_img_feats,
          local_text_feats, centers, text_centers, memory_feats,
          position, memory_vid, interpret=False):
    B, d = global_img_feat.shape
    P, M, _ = centers.shape
    vidf = memory_vid.astype(jnp.float32).reshape(1, M)
    pos2d = position.reshape(B, 1)

    nb1 = 8
    mb1 = M // nb1
    sim, align = pl.pallas_call(
        _k1_body,
        grid=(nb1,),
        in_specs=[
            pl.BlockSpec((B, d), lambda m: (0, 0)),
            pl.BlockSpec((B, d), lambda m: (0, 0)),
            pl.BlockSpec((P, B, d), lambda m: (0, 0, 0)),
            pl.BlockSpec((mb1, d), lambda m: (m, 0)),
        ],
        out_specs=[
            pl.BlockSpec((B, mb1), lambda m: (0, m)),
            pl.BlockSpec((1, 1), lambda m: (0, 0)),
        ],
        out_shape=[jax.ShapeDtypeStruct((B, M), jnp.float32),
                   jax.ShapeDtypeStruct((1, 1), jnp.float32)],
        scratch_shapes=[pltpu.VMEM((B, d), jnp.float32)],
        interpret=interpret,
    )(global_img_feat, global_text_feat, local_text_feats, memory_feats)

    idx, pv, mask = pl.pallas_call(
        _k2_body,
        out_shape=[jax.ShapeDtypeStruct((B, 16), jnp.int32),
                   jax.ShapeDtypeStruct((B, 16), jnp.int32),
                   jax.ShapeDtypeStruct((1, M), jnp.float32)],
        interpret=interpret,
    )(sim, vidf, pos2d)

    nblk = 8
    blk = M // nblk
    total, local = pl.pallas_call(
        _k3_body,
        grid=(nblk, P),
        in_specs=[
            pl.BlockSpec((1, B, d), lambda m, p: (p, 0, 0)),
            pl.BlockSpec((1, blk, d), lambda m, p: (p, m, 0)),
            pl.BlockSpec((1, blk), lambda m, p: (0, m)),
            pl.BlockSpec((B, 16), lambda m, p: (0, 0)),
            pl.BlockSpec((1, 1), lambda m, p: (0, 0)),
        ],
        out_specs=[
            pl.BlockSpec((1, 1), lambda m, p: (0, 0)),
            pl.BlockSpec((1, 1), lambda m, p: (0, 0)),
        ],
        out_shape=[jax.ShapeDtypeStruct((1, 1), jnp.float32),
                   jax.ShapeDtypeStruct((1, 1), jnp.float32)],
        scratch_shapes=[pltpu.VMEM((1, P * B), jnp.float32),
                        pltpu.VMEM((1, P * B), jnp.float32),
                        pltpu.VMEM((B, blk), jnp.float32)],
        interpret=interpret,
    )(local_img_feats, centers, mask, idx, align)

    return (total[0, 0], local[0, 0], align[0, 0], pv[:, :_KTOP])


def kernel(global_img_feat, global_text_feat, local_img_feats,
           local_text_feats, centers, text_centers, memory_feats,
           position, memory_vid):
    return _impl(global_img_feat, global_text_feat, local_img_feats,
                 local_text_feats, centers, text_centers, memory_feats,
                 position, memory_vid)
